# baseline (speedup 1.0000x reference)
import jax
import jax.numpy as jnp
from jax.experimental import pallas as pl
from jax.experimental.pallas import tpu as pltpu

SIZE = 1024
D = 10000
MODELS = 64
NOVELTY = 0.1
BD = 400


def _reghd_kernel(x_ref, w_ref, b_ref, clt_ref, alt_ref,
                  mr_ref, enc_ref, idx_ref, nov_ref,
                  sims_acc, adot_acc, clnsq_acc, encnsq_acc):
    i = pl.program_id(0)

    @pl.when(i == 0)
    def _init():
        sims_acc[...] = jnp.zeros_like(sims_acc)
        adot_acc[...] = jnp.zeros_like(adot_acc)
        clnsq_acc[...] = jnp.zeros_like(clnsq_acc)
        encnsq_acc[...] = jnp.zeros_like(encnsq_acc)

    e0 = x_ref[...] * w_ref[...]
    e1 = jnp.cos(e0 + b_ref[...]) * jnp.sin(e0)
    hv = jnp.sum(e1, axis=1, keepdims=True)
    enc = jnp.floor((hv + SIZE) / SIZE)
    enc_ref[...] = enc

    clt = clt_ref[...]
    alt = alt_ref[...]
    sims_acc[...] += jnp.sum(clt * enc, axis=0, keepdims=True)
    adot_acc[...] += jnp.sum(alt * enc, axis=0, keepdims=True)
    clnsq_acc[...] += jnp.sum(clt * clt, axis=0, keepdims=True)
    encnsq_acc[...] += jnp.sum(enc * enc, axis=(0, 1), keepdims=True)

    @pl.when(i == pl.num_programs(0) - 1)
    def _fin():
        sims = sims_acc[...] / (
            jnp.sqrt(clnsq_acc[...]) * jnp.sqrt(encnsq_acc[...]))
        mx = jnp.max(sims)
        iota = jax.lax.broadcasted_iota(jnp.int32, (1, MODELS), 1)
        idx = jnp.min(jnp.where(sims == mx, iota, MODELS))
        idx_ref[...] = jnp.full((1, 1), idx, jnp.int32)
        nov_ref[...] = jnp.all(sims < (1.0 - NOVELTY)).astype(
            jnp.int32).reshape(1, 1)
        mr_ref[...] = jnp.sum(
            jnp.where(iota == idx, adot_acc[...], 0.0)).reshape(1, 1)


def kernel(x, weight, bias, cluster, alpha, ts):
    x2 = x.reshape(1, SIZE)
    clt = cluster.T
    alt = alpha.T
    grid = (D // BD,)
    mr, enc, idx, nov = pl.pallas_call(
        _reghd_kernel,
        grid=grid,
        in_specs=[
            pl.BlockSpec((1, SIZE), lambda i: (0, 0)),
            pl.BlockSpec((BD, SIZE), lambda i: (i, 0)),
            pl.BlockSpec((BD, SIZE), lambda i: (i, 0)),
            pl.BlockSpec((BD, MODELS), lambda i: (i, 0)),
            pl.BlockSpec((BD, MODELS), lambda i: (i, 0)),
        ],
        out_specs=[
            pl.BlockSpec((1, 1), lambda i: (0, 0)),
            pl.BlockSpec((BD, 1), lambda i: (i, 0)),
            pl.BlockSpec((1, 1), lambda i: (0, 0)),
            pl.BlockSpec((1, 1), lambda i: (0, 0)),
        ],
        out_shape=[
            jax.ShapeDtypeStruct((1, 1), jnp.float32),
            jax.ShapeDtypeStruct((D, 1), jnp.float32),
            jax.ShapeDtypeStruct((1, 1), jnp.int32),
            jax.ShapeDtypeStruct((1, 1), jnp.int32),
        ],
        scratch_shapes=[
            pltpu.VMEM((1, MODELS), jnp.float32),
            pltpu.VMEM((1, MODELS), jnp.float32),
            pltpu.VMEM((1, MODELS), jnp.float32),
            pltpu.VMEM((1, 1), jnp.float32),
        ],
        compiler_params=pltpu.CompilerParams(
            dimension_semantics=("arbitrary",)),
    )(x2, weight, bias, clt, alt)
    return (mr.reshape(1), enc.reshape(D), idx.reshape(()),
            nov.reshape(()).astype(bool))


# speedup vs baseline: 3.3570x; 3.3570x over previous
import jax
import jax.numpy as jnp
from jax.experimental import pallas as pl
from jax.experimental.pallas import tpu as pltpu

SIZE = 1024
D = 10000
MODELS = 64
NOVELTY = 0.1
BD = 400

_INV2PI = 0.15915494309189535
_RC1 = 6.28125
_RC2 = 0.0019353072
_RC3 = 1.0253132e-11
_SIN_C = (0.9999999378189043, -0.16666621108236432, 0.008332791502750542,
          -0.0001981763098880802, 2.708831159301462e-06,
          -2.069813468752228e-08)
_COS_C = (0.9999999890921586, -0.4999998910158115, 0.04166648921768811,
          -0.0013887803597755755, 2.4769883554952905e-05,
          -2.7079030839512246e-07, 1.724509147595252e-09)


def _reduce_2pi(t):
    k = jax.lax.round(t * _INV2PI, jax.lax.RoundingMethod.TO_NEAREST_EVEN)
    f = t - k * _RC1
    f = f - k * _RC2
    f = f - k * _RC3
    return f


def _poly(c, x2):
    acc = jnp.full_like(x2, c[-1])
    for ci in c[-2::-1]:
        acc = acc * x2 + ci
    return acc


def _fast_sin(t):
    f = _reduce_2pi(t)
    return f * _poly(_SIN_C, f * f)


def _fast_cos(t):
    f = _reduce_2pi(t)
    return _poly(_COS_C, f * f)


def _reghd_kernel(x_ref, w_ref, b_ref, clt_ref, alt_ref,
                  mr_ref, enc_ref, idx_ref, nov_ref,
                  sims_acc, adot_acc, clnsq_acc, encnsq_acc):
    i = pl.program_id(0)

    @pl.when(i == 0)
    def _init():
        sims_acc[...] = jnp.zeros_like(sims_acc)
        adot_acc[...] = jnp.zeros_like(adot_acc)
        clnsq_acc[...] = jnp.zeros_like(clnsq_acc)
        encnsq_acc[...] = jnp.zeros_like(encnsq_acc)

    e0 = x_ref[...] * w_ref[...]
    e1 = _fast_cos(e0 + b_ref[...]) * _fast_sin(e0)
    hv = jnp.sum(e1, axis=1, keepdims=True)
    enc = jnp.floor((hv + SIZE) / SIZE)
    enc_ref[...] = enc

    clt = clt_ref[...]
    alt = alt_ref[...]
    sims_acc[...] += jnp.sum(clt * enc, axis=0, keepdims=True)
    adot_acc[...] += jnp.sum(alt * enc, axis=0, keepdims=True)
    clnsq_acc[...] += jnp.sum(clt * clt, axis=0, keepdims=True)
    encnsq_acc[...] += jnp.sum(enc * enc, axis=(0, 1), keepdims=True)

    @pl.when(i == pl.num_programs(0) - 1)
    def _fin():
        sims = sims_acc[...] / (
            jnp.sqrt(clnsq_acc[...]) * jnp.sqrt(encnsq_acc[...]))
        mx = jnp.max(sims)
        iota = jax.lax.broadcasted_iota(jnp.int32, (1, MODELS), 1)
        idx = jnp.min(jnp.where(sims == mx, iota, MODELS))
        idx_ref[...] = jnp.full((1, 1), idx, jnp.int32)
        nov_ref[...] = jnp.all(sims < (1.0 - NOVELTY)).astype(
            jnp.int32).reshape(1, 1)
        mr_ref[...] = jnp.sum(
            jnp.where(iota == idx, adot_acc[...], 0.0)).reshape(1, 1)


def kernel(x, weight, bias, cluster, alpha, ts):
    x2 = x.reshape(1, SIZE)
    clt = cluster.T
    alt = alpha.T
    grid = (D // BD,)
    mr, enc, idx, nov = pl.pallas_call(
        _reghd_kernel,
        grid=grid,
        in_specs=[
            pl.BlockSpec((1, SIZE), lambda i: (0, 0)),
            pl.BlockSpec((BD, SIZE), lambda i: (i, 0)),
            pl.BlockSpec((BD, SIZE), lambda i: (i, 0)),
            pl.BlockSpec((BD, MODELS), lambda i: (i, 0)),
            pl.BlockSpec((BD, MODELS), lambda i: (i, 0)),
        ],
        out_specs=[
            pl.BlockSpec((1, 1), lambda i: (0, 0)),
            pl.BlockSpec((BD, 1), lambda i: (i, 0)),
            pl.BlockSpec((1, 1), lambda i: (0, 0)),
            pl.BlockSpec((1, 1), lambda i: (0, 0)),
        ],
        out_shape=[
            jax.ShapeDtypeStruct((1, 1), jnp.float32),
            jax.ShapeDtypeStruct((D, 1), jnp.float32),
            jax.ShapeDtypeStruct((1, 1), jnp.int32),
            jax.ShapeDtypeStruct((1, 1), jnp.int32),
        ],
        scratch_shapes=[
            pltpu.VMEM((1, MODELS), jnp.float32),
            pltpu.VMEM((1, MODELS), jnp.float32),
            pltpu.VMEM((1, MODELS), jnp.float32),
            pltpu.VMEM((1, 1), jnp.float32),
        ],
        compiler_params=pltpu.CompilerParams(
            dimension_semantics=("arbitrary",)),
    )(x2, weight, bias, clt, alt)
    return (mr.reshape(1), enc.reshape(D), idx.reshape(()),
            nov.reshape(()).astype(bool))


# speedup vs baseline: 3.5218x; 1.0491x over previous
import jax
import jax.numpy as jnp
from jax.experimental import pallas as pl
from jax.experimental.pallas import tpu as pltpu

SIZE = 1024
D = 10000
MODELS = 64
NOVELTY = 0.1
BD = 1000

_INV2PI = 0.15915494309189535
_RC1 = 6.28125
_RC2 = 0.0019353072
_RC3 = 1.0253132e-11
_SIN_C = (0.9999999378189043, -0.16666621108236432, 0.008332791502750542,
          -0.0001981763098880802, 2.708831159301462e-06,
          -2.069813468752228e-08)
_COS_C = (0.9999999890921586, -0.4999998910158115, 0.04166648921768811,
          -0.0013887803597755755, 2.4769883554952905e-05,
          -2.7079030839512246e-07, 1.724509147595252e-09)


def _reduce_2pi(t):
    k = jax.lax.round(t * _INV2PI, jax.lax.RoundingMethod.TO_NEAREST_EVEN)
    f = t - k * _RC1
    f = f - k * _RC2
    return f


def _poly(c, x2):
    acc = jnp.full_like(x2, c[-1])
    for ci in c[-2::-1]:
        acc = acc * x2 + ci
    return acc


def _fast_sin(t):
    f = _reduce_2pi(t)
    return f * _poly(_SIN_C, f * f)


def _fast_cos(t):
    f = _reduce_2pi(t)
    return _poly(_COS_C, f * f)


def _reghd_kernel(x_ref, w_ref, b_ref, clt_ref, alt_ref,
                  mr_ref, enc_ref, idx_ref, nov_ref,
                  sims_acc, adot_acc, clnsq_acc, encnsq_acc):
    i = pl.program_id(0)

    @pl.when(i == 0)
    def _init():
        sims_acc[...] = jnp.zeros_like(sims_acc)
        adot_acc[...] = jnp.zeros_like(adot_acc)
        clnsq_acc[...] = jnp.zeros_like(clnsq_acc)
        encnsq_acc[...] = jnp.zeros_like(encnsq_acc)

    e0 = x_ref[...] * w_ref[...]
    e1 = _fast_cos(e0 + b_ref[...]) * _fast_sin(e0)
    hv = jnp.sum(e1, axis=1, keepdims=True)
    enc = jnp.floor((hv + SIZE) / SIZE)
    enc_ref[...] = enc

    clt = clt_ref[...]
    alt = alt_ref[...]
    sims_acc[...] += jnp.sum(clt * enc, axis=0, keepdims=True)
    adot_acc[...] += jnp.sum(alt * enc, axis=0, keepdims=True)
    clnsq_acc[...] += jnp.sum(clt * clt, axis=0, keepdims=True)
    encnsq_acc[...] += jnp.sum(enc * enc, axis=(0, 1), keepdims=True)

    @pl.when(i == pl.num_programs(0) - 1)
    def _fin():
        sims = sims_acc[...] / (
            jnp.sqrt(clnsq_acc[...]) * jnp.sqrt(encnsq_acc[...]))
        mx = jnp.max(sims)
        iota = jax.lax.broadcasted_iota(jnp.int32, (1, MODELS), 1)
        idx = jnp.min(jnp.where(sims == mx, iota, MODELS))
        idx_ref[...] = jnp.full((1, 1), idx, jnp.int32)
        nov_ref[...] = jnp.all(sims < (1.0 - NOVELTY)).astype(
            jnp.int32).reshape(1, 1)
        mr_ref[...] = jnp.sum(
            jnp.where(iota == idx, adot_acc[...], 0.0)).reshape(1, 1)


def kernel(x, weight, bias, cluster, alpha, ts):
    x2 = x.reshape(1, SIZE)
    clt = cluster.T
    alt = alpha.T
    grid = (D // BD,)
    mr, enc, idx, nov = pl.pallas_call(
        _reghd_kernel,
        grid=grid,
        in_specs=[
            pl.BlockSpec((1, SIZE), lambda i: (0, 0)),
            pl.BlockSpec((BD, SIZE), lambda i: (i, 0)),
            pl.BlockSpec((BD, SIZE), lambda i: (i, 0)),
            pl.BlockSpec((BD, MODELS), lambda i: (i, 0)),
            pl.BlockSpec((BD, MODELS), lambda i: (i, 0)),
        ],
        out_specs=[
            pl.BlockSpec((1, 1), lambda i: (0, 0)),
            pl.BlockSpec((BD, 1), lambda i: (i, 0)),
            pl.BlockSpec((1, 1), lambda i: (0, 0)),
            pl.BlockSpec((1, 1), lambda i: (0, 0)),
        ],
        out_shape=[
            jax.ShapeDtypeStruct((1, 1), jnp.float32),
            jax.ShapeDtypeStruct((D, 1), jnp.float32),
            jax.ShapeDtypeStruct((1, 1), jnp.int32),
            jax.ShapeDtypeStruct((1, 1), jnp.int32),
        ],
        scratch_shapes=[
            pltpu.VMEM((1, MODELS), jnp.float32),
            pltpu.VMEM((1, MODELS), jnp.float32),
            pltpu.VMEM((1, MODELS), jnp.float32),
            pltpu.VMEM((1, 1), jnp.float32),
        ],
        compiler_params=pltpu.CompilerParams(
            dimension_semantics=("arbitrary",)),
    )(x2, weight, bias, clt, alt)
    return (mr.reshape(1), enc.reshape(D), idx.reshape(()),
            nov.reshape(()).astype(bool))


# speedup vs baseline: 3.8223x; 1.0853x over previous
import jax
import jax.numpy as jnp
from jax.experimental import pallas as pl
from jax.experimental.pallas import tpu as pltpu

SIZE = 1024
D = 10000
MODELS = 64
NOVELTY = 0.1
BD = 1000

_INV2PI = 0.15915494309189535
_RC1 = 6.28125
_RC2 = 0.0019353072
_PI = 3.141592653589793
_HSIN_C = (0.5 * 0.9999999378189043, 0.5 * -0.16666621108236432,
           0.5 * 0.008332791502750542, 0.5 * -0.0001981763098880802,
           0.5 * 2.708831159301462e-06, 0.5 * -2.069813468752228e-08)


def _half_sin_nored(f):
    x2 = f * f
    acc = jnp.full_like(x2, _HSIN_C[-1])
    for ci in _HSIN_C[-2::-1]:
        acc = acc * x2 + ci
    return f * acc


def _half_sin(t):
    k = jax.lax.round(t * _INV2PI, jax.lax.RoundingMethod.TO_NEAREST_EVEN)
    f = t - k * _RC1
    f = f - k * _RC2
    return _half_sin_nored(f)


def _reghd_kernel(x_ref, w_ref, b_ref, clt_ref, alt_ref,
                  mr_ref, enc_ref, idx_ref, nov_ref,
                  sims_acc, adot_acc, clnsq_acc, encnsq_acc):
    i = pl.program_id(0)

    @pl.when(i == 0)
    def _init():
        sims_acc[...] = jnp.zeros_like(sims_acc)
        adot_acc[...] = jnp.zeros_like(adot_acc)
        clnsq_acc[...] = jnp.zeros_like(clnsq_acc)
        encnsq_acc[...] = jnp.zeros_like(encnsq_acc)

    e0 = x_ref[...] * w_ref[...]
    b = b_ref[...]
    e1 = _half_sin(e0 + e0 + b) + _half_sin_nored(b - _PI)
    hv = jnp.sum(e1, axis=1, keepdims=True)
    enc = jnp.floor((hv + SIZE) / SIZE)
    enc_ref[...] = enc

    clt = clt_ref[...]
    alt = alt_ref[...]
    sims_acc[...] += jnp.sum(clt * enc, axis=0, keepdims=True)
    adot_acc[...] += jnp.sum(alt * enc, axis=0, keepdims=True)
    clnsq_acc[...] += jnp.sum(clt * clt, axis=0, keepdims=True)
    encnsq_acc[...] += jnp.sum(enc * enc, axis=(0, 1), keepdims=True)

    @pl.when(i == pl.num_programs(0) - 1)
    def _fin():
        sims = sims_acc[...] / (
            jnp.sqrt(clnsq_acc[...]) * jnp.sqrt(encnsq_acc[...]))
        mx = jnp.max(sims)
        iota = jax.lax.broadcasted_iota(jnp.int32, (1, MODELS), 1)
        idx = jnp.min(jnp.where(sims == mx, iota, MODELS))
        idx_ref[...] = jnp.full((1, 1), idx, jnp.int32)
        nov_ref[...] = jnp.all(sims < (1.0 - NOVELTY)).astype(
            jnp.int32).reshape(1, 1)
        mr_ref[...] = jnp.sum(
            jnp.where(iota == idx, adot_acc[...], 0.0)).reshape(1, 1)


def kernel(x, weight, bias, cluster, alpha, ts):
    x2 = x.reshape(1, SIZE)
    clt = cluster.T
    alt = alpha.T
    grid = (D // BD,)
    mr, enc, idx, nov = pl.pallas_call(
        _reghd_kernel,
        grid=grid,
        in_specs=[
            pl.BlockSpec((1, SIZE), lambda i: (0, 0)),
            pl.BlockSpec((BD, SIZE), lambda i: (i, 0)),
            pl.BlockSpec((BD, SIZE), lambda i: (i, 0)),
            pl.BlockSpec((BD, MODELS), lambda i: (i, 0)),
            pl.BlockSpec((BD, MODELS), lambda i: (i, 0)),
        ],
        out_specs=[
            pl.BlockSpec((1, 1), lambda i: (0, 0)),
            pl.BlockSpec((BD, 1), lambda i: (i, 0)),
            pl.BlockSpec((1, 1), lambda i: (0, 0)),
            pl.BlockSpec((1, 1), lambda i: (0, 0)),
        ],
        out_shape=[
            jax.ShapeDtypeStruct((1, 1), jnp.float32),
            jax.ShapeDtypeStruct((D, 1), jnp.float32),
            jax.ShapeDtypeStruct((1, 1), jnp.int32),
            jax.ShapeDtypeStruct((1, 1), jnp.int32),
        ],
        scratch_shapes=[
            pltpu.VMEM((1, MODELS), jnp.float32),
            pltpu.VMEM((1, MODELS), jnp.float32),
            pltpu.VMEM((1, MODELS), jnp.float32),
            pltpu.VMEM((1, 1), jnp.float32),
        ],
        compiler_params=pltpu.CompilerParams(
            dimension_semantics=("arbitrary",)),
    )(x2, weight, bias, clt, alt)
    return (mr.reshape(1), enc.reshape(D), idx.reshape(()),
            nov.reshape(()).astype(bool))
